# Initial kernel scaffold; baseline (speedup 1.0000x reference)
import jax
import jax.numpy as jnp
from jax.experimental import pallas as pl


def kernel(h, adj, W, a, tensor1, tensor2):
    raise NotImplementedError("write your pallas kernel here")


# speedup vs baseline: 19.8142x; 19.8142x over previous
import functools

import jax
import jax.numpy as jnp
from jax.experimental import pallas as pl
from jax.experimental.pallas import tpu as pltpu

N = 4096
IN_F = 512
OUT_F = 512
ALPHA = 0.2
KC = 3
C_SMALL = -9e-15
D_BIG = -9e15

NPAD = 4352
BM = 256
BN1 = 1024


def _secondmin_body(hi_ref, hj_ref, sqi_ref, sqj_ref, out_ref, m1_ref, m2_ref):
    j = pl.program_id(1)
    nj = pl.num_programs(1)

    @pl.when(j == 0)
    def _():
        m1_ref[...] = jnp.full_like(m1_ref, jnp.inf)
        m2_ref[...] = jnp.full_like(m2_ref, jnp.inf)

    dots = jax.lax.dot_general(
        hi_ref[...], hj_ref[...], (((1,), (1,)), ((), ())),
        preferred_element_type=jnp.float32)
    d2 = sqi_ref[...] + sqj_ref[...] - 2.0 * dots
    d = jnp.sqrt(jnp.maximum(d2, 0.0))

    t1 = jnp.min(d, axis=1, keepdims=True)
    eq = d == t1
    cnt = jnp.sum(eq.astype(jnp.float32), axis=1, keepdims=True)
    t2_distinct = jnp.min(jnp.where(eq, jnp.inf, d), axis=1, keepdims=True)
    t2 = jnp.where(cnt > 1.0, t1, t2_distinct)

    r1 = m1_ref[...]
    r2 = m2_ref[...]
    m1_ref[...] = jnp.minimum(r1, t1)
    m2_ref[...] = jnp.minimum(jnp.maximum(r1, t1), jnp.minimum(r2, t2))

    @pl.when(j == nj - 1)
    def _():
        out_ref[...] = m2_ref[...]


def _secondmin(h, sq_col, sq_row):
    ni, nj = N // BM, N // BN1
    return pl.pallas_call(
        _secondmin_body,
        grid=(ni, nj),
        in_specs=[
            pl.BlockSpec((BM, IN_F), lambda i, j: (i, 0)),
            pl.BlockSpec((BN1, IN_F), lambda i, j: (j, 0)),
            pl.BlockSpec((BM, 1), lambda i, j: (i, 0)),
            pl.BlockSpec((1, BN1), lambda i, j: (0, j)),
        ],
        out_specs=pl.BlockSpec((BM, 1), lambda i, j: (i, 0)),
        out_shape=jax.ShapeDtypeStruct((N, 1), jnp.float32),
        scratch_shapes=[
            pltpu.VMEM((BM, 1), jnp.float32),
            pltpu.VMEM((BM, 1), jnp.float32),
        ],
    )(h, h, sq_col, sq_row)


def _proj_body(h_ref, w_ref, a1_ref, a2_ref, wh_ref, wh1_ref, wh2_ref):
    wh = jnp.dot(h_ref[...], w_ref[...], preferred_element_type=jnp.float32)
    wh_ref[...] = wh
    wh1_ref[...] = jnp.dot(wh, a1_ref[...], preferred_element_type=jnp.float32)
    wh2_ref[...] = jnp.dot(wh, a2_ref[...], preferred_element_type=jnp.float32)


def _proj(h, W, a1, a2):
    ni = N // BM
    return pl.pallas_call(
        _proj_body,
        grid=(ni,),
        in_specs=[
            pl.BlockSpec((BM, IN_F), lambda i: (i, 0)),
            pl.BlockSpec((IN_F, OUT_F), lambda i: (0, 0)),
            pl.BlockSpec((OUT_F, 1), lambda i: (0, 0)),
            pl.BlockSpec((OUT_F, 1), lambda i: (0, 0)),
        ],
        out_specs=[
            pl.BlockSpec((BM, OUT_F), lambda i: (i, 0)),
            pl.BlockSpec((BM, 1), lambda i: (i, 0)),
            pl.BlockSpec((BM, 1), lambda i: (i, 0)),
        ],
        out_shape=[
            jax.ShapeDtypeStruct((N, OUT_F), jnp.float32),
            jax.ShapeDtypeStruct((N, 1), jnp.float32),
            jax.ShapeDtypeStruct((N, 1), jnp.float32),
        ],
    )(h, W, a1, a2)


def _kmeans_body(h_ref, d1_ref):
    h = h_ref[...]

    def dist2(c_row):
        diff = h - c_row
        return jnp.sum(diff * diff, axis=1, keepdims=True)

    def step(_, centers):
        d20 = dist2(centers[0:1, :])
        d21 = dist2(centers[1:2, :])
        d22 = dist2(centers[2:3, :])
        m01 = jnp.minimum(d20, d21)
        is2 = d22 < m01
        is1 = jnp.logical_and(jnp.logical_not(is2), d21 < d20)
        is0 = jnp.logical_and(jnp.logical_not(is2), jnp.logical_not(is1))

        def upd(mask):
            sums = jnp.sum(jnp.where(mask, h, 0.0), axis=0, keepdims=True)
            cnt = jnp.sum(mask.astype(jnp.float32))
            return sums / jnp.maximum(cnt, 1.0)

        return jnp.concatenate(
            [upd(is0), upd(is1), upd(is2), jnp.zeros((5, IN_F), jnp.float32)],
            axis=0)

    centers = jax.lax.fori_loop(0, 10, step, h_ref[0:8, :])
    d1cols = [jnp.sqrt(dist2(centers[k:k + 1, :])) for k in range(KC)]
    d1_ref[...] = jnp.concatenate(
        d1cols + [jnp.zeros((N, 8 - KC), jnp.float32)], axis=1)


def _kmeans_d1(h):
    return pl.pallas_call(
        _kmeans_body,
        in_specs=[pl.BlockSpec((N, IN_F), lambda: (0, 0))],
        out_specs=pl.BlockSpec((N, 8), lambda: (0, 0)),
        out_shape=jax.ShapeDtypeStruct((N, 8), jnp.float32),
    )(h)


def _colsum_body(wh1_ref, wh2t_ref, w4_ref, out_ref):
    BI = 512
    acc = jnp.zeros((8, wh2t_ref.shape[1]), jnp.float32)
    wh2t = wh2t_ref[...]
    for i in range(N // BI):
        wh1t = wh1_ref[pl.ds(i * BI, BI), :]
        x = wh1t + wh2t
        e = jnp.where(x >= 0.0, x, ALPHA * x)
        w4t = w4_ref[pl.ds(i * BI, BI), :]
        acc = acc + jax.lax.dot_general(
            w4t, e, (((0,), (0,)), ((), ())),
            preferred_element_type=jnp.float32)
    out_ref[...] = acc


def _colsums(wh1, wh2t, w4):
    BNC = 512
    return pl.pallas_call(
        _colsum_body,
        grid=(N // BNC,),
        in_specs=[
            pl.BlockSpec((N, 1), lambda j: (0, 0)),
            pl.BlockSpec((1, BNC), lambda j: (0, j)),
            pl.BlockSpec((N, 8), lambda j: (0, 0)),
        ],
        out_specs=pl.BlockSpec((8, BNC), lambda j: (0, j)),
        out_shape=jax.ShapeDtypeStruct((8, N), jnp.float32),
    )(wh1, wh2t, w4)


def _flash_body(u_ref, v_ref, whp_ref, out_ref, acc_ref, m_ref, l_ref):
    j = pl.program_id(1)
    nj = pl.num_programs(1)

    @pl.when(j == 0)
    def _():
        acc_ref[...] = jnp.zeros_like(acc_ref)
        m_ref[...] = jnp.full_like(m_ref, -jnp.inf)
        l_ref[...] = jnp.zeros_like(l_ref)

    logits = jnp.dot(u_ref[...], v_ref[...], preferred_element_type=jnp.float32)
    m_prev = m_ref[...]
    m_new = jnp.maximum(m_prev, jnp.max(logits, axis=1, keepdims=True))
    scale = jnp.exp(m_prev - m_new)
    p = jnp.exp(logits - m_new)
    l_ref[...] = l_ref[...] * scale + jnp.sum(p, axis=1, keepdims=True)
    acc_ref[...] = acc_ref[...] * scale + jnp.dot(
        p, whp_ref[...], preferred_element_type=jnp.float32)
    m_ref[...] = m_new

    @pl.when(j == nj - 1)
    def _():
        hp = acc_ref[...] / l_ref[...]
        out_ref[...] = jnp.where(hp > 0.0, hp, jnp.exp(hp) - 1.0)


def _flash(u, v, whp):
    nb = NPAD // BM
    return pl.pallas_call(
        _flash_body,
        grid=(nb, nb),
        in_specs=[
            pl.BlockSpec((BM, 8), lambda i, j: (i, 0)),
            pl.BlockSpec((8, BM), lambda i, j: (0, j)),
            pl.BlockSpec((BM, OUT_F), lambda i, j: (j, 0)),
        ],
        out_specs=pl.BlockSpec((BM, OUT_F), lambda i, j: (i, 0)),
        out_shape=jax.ShapeDtypeStruct((NPAD, OUT_F), jnp.float32),
        scratch_shapes=[
            pltpu.VMEM((BM, OUT_F), jnp.float32),
            pltpu.VMEM((BM, 1), jnp.float32),
            pltpu.VMEM((BM, 1), jnp.float32),
        ],
    )(u, v, whp)


def kernel(h, adj, W, a, tensor1, tensor2):
    del adj
    f32 = jnp.float32
    h = h.astype(f32)

    sq = jnp.sum(h * h, axis=1)
    m2 = _secondmin(h, sq.reshape(N, 1), sq.reshape(1, N))
    dc = jnp.mean(m2)

    d1 = _kmeans_d1(h)[:, :KC]
    near2 = jnp.take_along_axis(
        d1, jnp.clip(jnp.arange(N)[:, None] * KC + 1, 0, KC - 1), axis=1)
    d1b = jnp.where(d1 != 0.0, dc * near2 / (d1 * d1), d1) - 9e-15

    wh, wh1, wh2 = _proj(h, W, a[:OUT_F, :], a[OUT_F:, :])

    w4 = jnp.concatenate(
        [jnp.ones((N, 1), f32), d1b, jnp.zeros((N, 8 - 1 - KC), f32)], axis=1)
    sce = _colsums(wh1, wh2.reshape(1, N), w4)
    s = sce[0, :]
    ce = sce[1:1 + KC, :]

    sigma1 = jnp.sum(tensor1, axis=0)
    t2s = jnp.sum(tensor2, axis=0)
    ct1 = d1b.T @ tensor1

    npad_tail = NPAD - (N + KC)
    c = f32(C_SMALL)
    d = f32(D_BIG)

    v_row0 = jnp.concatenate(
        [c * s, c * sigma1, jnp.full((npad_tail,), -1e30, f32)])
    v_mid = jnp.concatenate(
        [tensor2, jnp.zeros((KC, KC + npad_tail), f32)], axis=1)
    v_bot = jnp.concatenate(
        [ce + d * t2s[None, :], ct1, jnp.zeros((KC, npad_tail), f32)], axis=1)
    v = jnp.concatenate(
        [v_row0[None, :], v_mid, v_bot, jnp.zeros((1, NPAD), f32)], axis=0)

    u_top = jnp.concatenate(
        [jnp.ones((N, 1), f32), d1b, jnp.zeros((N, 4), f32)], axis=1)
    u_bot = jnp.concatenate(
        [jnp.ones((KC, 1), f32), jnp.zeros((KC, KC), f32),
         jnp.eye(KC, dtype=f32), jnp.zeros((KC, 1), f32)], axis=1)
    u = jnp.concatenate(
        [u_top, u_bot, jnp.zeros((npad_tail, 8), f32)], axis=0)

    whp = jnp.concatenate(
        [wh, jnp.zeros((NPAD - N, OUT_F), f32)], axis=0)

    out = _flash(u, v, whp)
    return out[:N + KC, :]


# speedup vs baseline: 28.4554x; 1.4361x over previous
import functools

import jax
import jax.numpy as jnp
from jax.experimental import pallas as pl
from jax.experimental.pallas import tpu as pltpu

N = 4096
IN_F = 512
OUT_F = 512
ALPHA = 0.2
KC = 3
C_SMALL = -9e-15
D_BIG = -9e15

NPAD = 4608
BM = 256
BF = 512
BN1 = 1024


def _secondmin_body(hi_ref, hj_ref, sqi_ref, sqj_ref, out_ref, m1_ref, m2_ref):
    j = pl.program_id(1)
    nj = pl.num_programs(1)

    @pl.when(j == 0)
    def _():
        m1_ref[...] = jnp.full_like(m1_ref, jnp.inf)
        m2_ref[...] = jnp.full_like(m2_ref, jnp.inf)

    dots = jax.lax.dot_general(
        hi_ref[...], hj_ref[...], (((1,), (1,)), ((), ())),
        preferred_element_type=jnp.float32)
    d2 = sqi_ref[...] + sqj_ref[...] - 2.0 * dots
    d = jnp.sqrt(jnp.maximum(d2, 0.0))

    t1 = jnp.min(d, axis=1, keepdims=True)
    eq = d == t1
    cnt = jnp.sum(eq.astype(jnp.float32), axis=1, keepdims=True)
    t2_distinct = jnp.min(jnp.where(eq, jnp.inf, d), axis=1, keepdims=True)
    t2 = jnp.where(cnt > 1.0, t1, t2_distinct)

    r1 = m1_ref[...]
    r2 = m2_ref[...]
    m1_ref[...] = jnp.minimum(r1, t1)
    m2_ref[...] = jnp.minimum(jnp.maximum(r1, t1), jnp.minimum(r2, t2))

    @pl.when(j == nj - 1)
    def _():
        out_ref[...] = m2_ref[...]


def _secondmin(h, sq_col, sq_row):
    ni, nj = N // BM, N // BN1
    return pl.pallas_call(
        _secondmin_body,
        grid=(ni, nj),
        in_specs=[
            pl.BlockSpec((BM, IN_F), lambda i, j: (i, 0)),
            pl.BlockSpec((BN1, IN_F), lambda i, j: (j, 0)),
            pl.BlockSpec((BM, 1), lambda i, j: (i, 0)),
            pl.BlockSpec((1, BN1), lambda i, j: (0, j)),
        ],
        out_specs=pl.BlockSpec((BM, 1), lambda i, j: (i, 0)),
        out_shape=jax.ShapeDtypeStruct((N, 1), jnp.float32),
        scratch_shapes=[
            pltpu.VMEM((BM, 1), jnp.float32),
            pltpu.VMEM((BM, 1), jnp.float32),
        ],
    )(h, h, sq_col, sq_row)


def _proj_body(h_ref, w_ref, a1_ref, a2_ref, wh_ref, wh1_ref, wh2_ref):
    wh = jnp.dot(h_ref[...], w_ref[...], preferred_element_type=jnp.float32)
    wh_ref[...] = wh
    wh1_ref[...] = jnp.dot(wh, a1_ref[...], preferred_element_type=jnp.float32)
    wh2_ref[...] = jnp.dot(wh, a2_ref[...], preferred_element_type=jnp.float32)


def _proj(h, W, a1, a2):
    ni = N // BM
    return pl.pallas_call(
        _proj_body,
        grid=(ni,),
        in_specs=[
            pl.BlockSpec((BM, IN_F), lambda i: (i, 0)),
            pl.BlockSpec((IN_F, OUT_F), lambda i: (0, 0)),
            pl.BlockSpec((OUT_F, 1), lambda i: (0, 0)),
            pl.BlockSpec((OUT_F, 1), lambda i: (0, 0)),
        ],
        out_specs=[
            pl.BlockSpec((BM, OUT_F), lambda i: (i, 0)),
            pl.BlockSpec((BM, 1), lambda i: (i, 0)),
            pl.BlockSpec((BM, 1), lambda i: (i, 0)),
        ],
        out_shape=[
            jax.ShapeDtypeStruct((N, OUT_F), jnp.float32),
            jax.ShapeDtypeStruct((N, 1), jnp.float32),
            jax.ShapeDtypeStruct((N, 1), jnp.float32),
        ],
    )(h, W, a1, a2)


def _kmeans_body(h_ref, d1_ref):
    TK = 512

    def step(_, centers):
        c0 = centers[0:1, :]
        c1 = centers[1:2, :]
        c2 = centers[2:3, :]
        s0 = jnp.zeros((1, IN_F), jnp.float32)
        s1 = jnp.zeros((1, IN_F), jnp.float32)
        s2 = jnp.zeros((1, IN_F), jnp.float32)
        n0 = jnp.float32(0.0)
        n1 = jnp.float32(0.0)
        n2 = jnp.float32(0.0)
        for t in range(N // TK):
            ht = h_ref[pl.ds(t * TK, TK), :]

            def dist2(c_row):
                diff = ht - c_row
                return jnp.sum(diff * diff, axis=1, keepdims=True)

            d20, d21, d22 = dist2(c0), dist2(c1), dist2(c2)
            m01 = jnp.minimum(d20, d21)
            is2 = d22 < m01
            is1 = jnp.logical_and(jnp.logical_not(is2), d21 < d20)
            is0 = jnp.logical_and(jnp.logical_not(is2), jnp.logical_not(is1))
            s0 = s0 + jnp.sum(jnp.where(is0, ht, 0.0), axis=0, keepdims=True)
            s1 = s1 + jnp.sum(jnp.where(is1, ht, 0.0), axis=0, keepdims=True)
            s2 = s2 + jnp.sum(jnp.where(is2, ht, 0.0), axis=0, keepdims=True)
            n0 = n0 + jnp.sum(is0.astype(jnp.float32))
            n1 = n1 + jnp.sum(is1.astype(jnp.float32))
            n2 = n2 + jnp.sum(is2.astype(jnp.float32))
        return jnp.concatenate(
            [s0 / jnp.maximum(n0, 1.0),
             s1 / jnp.maximum(n1, 1.0),
             s2 / jnp.maximum(n2, 1.0),
             jnp.zeros((5, IN_F), jnp.float32)], axis=0)

    centers = jax.lax.fori_loop(0, 10, step, h_ref[0:8, :])
    h = h_ref[...]

    def dist2_full(c_row):
        diff = h - c_row
        return jnp.sum(diff * diff, axis=1, keepdims=True)

    d1cols = [jnp.sqrt(dist2_full(centers[k:k + 1, :])) for k in range(KC)]
    d1_ref[...] = jnp.concatenate(
        d1cols + [jnp.zeros((N, 8 - KC), jnp.float32)], axis=1)


def _kmeans_d1(h):
    return pl.pallas_call(
        _kmeans_body,
        in_specs=[pl.BlockSpec((N, IN_F), lambda: (0, 0))],
        out_specs=pl.BlockSpec((N, 8), lambda: (0, 0)),
        out_shape=jax.ShapeDtypeStruct((N, 8), jnp.float32),
    )(h)


def _colsum_body(wh1_ref, wh2t_ref, w4_ref, out_ref):
    BI = 512
    acc = jnp.zeros((8, wh2t_ref.shape[1]), jnp.float32)
    wh2t = wh2t_ref[...]
    for i in range(N // BI):
        wh1t = wh1_ref[pl.ds(i * BI, BI), :]
        x = wh1t + wh2t
        e = jnp.where(x >= 0.0, x, ALPHA * x)
        w4t = w4_ref[pl.ds(i * BI, BI), :]
        acc = acc + jax.lax.dot_general(
            w4t, e, (((0,), (0,)), ((), ())),
            preferred_element_type=jnp.float32)
    out_ref[...] = acc


def _colsums(wh1, wh2t, w4):
    BNC = 512
    return pl.pallas_call(
        _colsum_body,
        grid=(N // BNC,),
        in_specs=[
            pl.BlockSpec((N, 1), lambda j: (0, 0)),
            pl.BlockSpec((1, BNC), lambda j: (0, j)),
            pl.BlockSpec((N, 8), lambda j: (0, 0)),
        ],
        out_specs=pl.BlockSpec((8, BNC), lambda j: (0, j)),
        out_shape=jax.ShapeDtypeStruct((8, N), jnp.float32),
    )(wh1, wh2t, w4)


def _flash_body(u_ref, v_ref, whp_ref, out_ref, acc_ref, m_ref, l_ref):
    j = pl.program_id(1)
    nj = pl.num_programs(1)

    @pl.when(j == 0)
    def _():
        acc_ref[...] = jnp.zeros_like(acc_ref)
        m_ref[...] = jnp.full_like(m_ref, -jnp.inf)
        l_ref[...] = jnp.zeros_like(l_ref)

    logits = jnp.dot(u_ref[...], v_ref[...], preferred_element_type=jnp.float32)
    m_prev = m_ref[...]
    m_new = jnp.maximum(m_prev, jnp.max(logits, axis=1, keepdims=True))
    scale = jnp.exp(m_prev - m_new)
    p = jnp.exp(logits - m_new)
    l_ref[...] = l_ref[...] * scale + jnp.sum(p, axis=1, keepdims=True)
    acc_ref[...] = acc_ref[...] * scale + jnp.dot(
        p, whp_ref[...], preferred_element_type=jnp.float32)
    m_ref[...] = m_new

    @pl.when(j == nj - 1)
    def _():
        hp = acc_ref[...] / l_ref[...]
        out_ref[...] = jnp.where(hp > 0.0, hp, jnp.exp(hp) - 1.0)


def _flash(u, v, whp):
    nb = NPAD // BF
    return pl.pallas_call(
        _flash_body,
        grid=(nb, nb),
        in_specs=[
            pl.BlockSpec((BF, 8), lambda i, j: (i, 0)),
            pl.BlockSpec((8, BF), lambda i, j: (0, j)),
            pl.BlockSpec((BF, OUT_F), lambda i, j: (j, 0)),
        ],
        out_specs=pl.BlockSpec((BF, OUT_F), lambda i, j: (i, 0)),
        out_shape=jax.ShapeDtypeStruct((NPAD, OUT_F), jnp.float32),
        scratch_shapes=[
            pltpu.VMEM((BF, OUT_F), jnp.float32),
            pltpu.VMEM((BF, 1), jnp.float32),
            pltpu.VMEM((BF, 1), jnp.float32),
        ],
    )(u, v, whp)


def kernel(h, adj, W, a, tensor1, tensor2):
    del adj
    f32 = jnp.float32
    h = h.astype(f32)

    sq = jnp.sum(h * h, axis=1)
    m2 = _secondmin(h, sq.reshape(N, 1), sq.reshape(1, N))
    dc = jnp.mean(m2)

    d1 = _kmeans_d1(h)[:, :KC]
    near2 = jnp.take_along_axis(
        d1, jnp.clip(jnp.arange(N)[:, None] * KC + 1, 0, KC - 1), axis=1)
    d1b = jnp.where(d1 != 0.0, dc * near2 / (d1 * d1), d1) - 9e-15

    wh, wh1, wh2 = _proj(h, W, a[:OUT_F, :], a[OUT_F:, :])

    w4 = jnp.concatenate(
        [jnp.ones((N, 1), f32), d1b, jnp.zeros((N, 8 - 1 - KC), f32)], axis=1)
    sce = _colsums(wh1, wh2.reshape(1, N), w4)
    s = sce[0, :]
    ce = sce[1:1 + KC, :]

    sigma1 = jnp.sum(tensor1, axis=0)
    t2s = jnp.sum(tensor2, axis=0)
    ct1 = d1b.T @ tensor1

    npad_tail = NPAD - (N + KC)
    c = f32(C_SMALL)
    d = f32(D_BIG)

    v_row0 = jnp.concatenate(
        [c * s, c * sigma1, jnp.full((npad_tail,), -1e30, f32)])
    v_mid = jnp.concatenate(
        [tensor2, jnp.zeros((KC, KC + npad_tail), f32)], axis=1)
    v_bot = jnp.concatenate(
        [ce + d * t2s[None, :], ct1, jnp.zeros((KC, npad_tail), f32)], axis=1)
    v = jnp.concatenate(
        [v_row0[None, :], v_mid, v_bot, jnp.zeros((1, NPAD), f32)], axis=0)

    u_top = jnp.concatenate(
        [jnp.ones((N, 1), f32), d1b, jnp.zeros((N, 4), f32)], axis=1)
    u_bot = jnp.concatenate(
        [jnp.ones((KC, 1), f32), jnp.zeros((KC, KC), f32),
         jnp.eye(KC, dtype=f32), jnp.zeros((KC, 1), f32)], axis=1)
    u = jnp.concatenate(
        [u_top, u_bot, jnp.zeros((npad_tail, 8), f32)], axis=0)

    whp = jnp.concatenate(
        [wh, jnp.zeros((NPAD - N, OUT_F), f32)], axis=0)

    out = _flash(u, v, whp)
    return out[:N + KC, :]


# speedup vs baseline: 35.7217x; 1.2554x over previous
import jax
import jax.numpy as jnp
from jax.experimental import pallas as pl
from jax.experimental.pallas import tpu as pltpu

N = 4096
IN_F = 512
OUT_F = 512
ALPHA = 0.2
KC = 3
C_SMALL = -9e-15
D_BIG = -9e15

NPAD = 4608
BM = 256
BF = 512
BN1 = 1024


def _secondmin_body(hi_ref, hj_ref, sqi_ref, sqj_ref, out_ref, m1_ref, m2_ref):
    j = pl.program_id(1)
    nj = pl.num_programs(1)

    @pl.when(j == 0)
    def _():
        m1_ref[...] = jnp.full_like(m1_ref, jnp.inf)
        m2_ref[...] = jnp.full_like(m2_ref, jnp.inf)

    hj = hj_ref[pl.ds(j * BN1, BN1), :]
    sqj = sqj_ref[:, pl.ds(j * BN1, BN1)]
    dots = jax.lax.dot_general(
        hi_ref[...], hj, (((1,), (1,)), ((), ())),
        preferred_element_type=jnp.float32)
    d2 = sqi_ref[...] + sqj - 2.0 * dots
    d = jnp.sqrt(jnp.maximum(d2, 0.0))

    t1 = jnp.min(d, axis=1, keepdims=True)
    eq = d == t1
    cnt = jnp.sum(eq.astype(jnp.float32), axis=1, keepdims=True)
    t2_distinct = jnp.min(jnp.where(eq, jnp.inf, d), axis=1, keepdims=True)
    t2 = jnp.where(cnt > 1.0, t1, t2_distinct)

    r1 = m1_ref[...]
    r2 = m2_ref[...]
    m1_ref[...] = jnp.minimum(r1, t1)
    m2_ref[...] = jnp.minimum(jnp.maximum(r1, t1), jnp.minimum(r2, t2))

    @pl.when(j == nj - 1)
    def _():
        out_ref[...] = m2_ref[...]


def _secondmin(h, sq_col, sq_row):
    ni, nj = N // BM, N // BN1
    return pl.pallas_call(
        _secondmin_body,
        grid=(ni, nj),
        in_specs=[
            pl.BlockSpec((BM, IN_F), lambda i, j: (i, 0)),
            pl.BlockSpec((N, IN_F), lambda i, j: (0, 0)),
            pl.BlockSpec((BM, 1), lambda i, j: (i, 0)),
            pl.BlockSpec((1, N), lambda i, j: (0, 0)),
        ],
        out_specs=pl.BlockSpec((BM, 1), lambda i, j: (i, 0)),
        out_shape=jax.ShapeDtypeStruct((N, 1), jnp.float32),
        scratch_shapes=[
            pltpu.VMEM((BM, 1), jnp.float32),
            pltpu.VMEM((BM, 1), jnp.float32),
        ],
    )(h, h, sq_col, sq_row)


def _proj_body(h_ref, w_ref, a1_ref, a2_ref, wh_ref, wh1_ref, wh2_ref):
    i = pl.program_id(0)

    @pl.when(i < N // BF)
    def _():
        wh = jnp.dot(h_ref[...], w_ref[...], preferred_element_type=jnp.float32)
        wh_ref[...] = wh
        wh1_ref[...] = jnp.dot(wh, a1_ref[...],
                               preferred_element_type=jnp.float32)
        wh2_ref[...] = jnp.dot(wh, a2_ref[...],
                               preferred_element_type=jnp.float32)

    @pl.when(i >= N // BF)
    def _():
        wh_ref[...] = jnp.zeros_like(wh_ref)


def _proj(h, W, a1, a2):
    nh = N // BF
    return pl.pallas_call(
        _proj_body,
        grid=(NPAD // BF,),
        in_specs=[
            pl.BlockSpec((BF, IN_F), lambda i: (jnp.minimum(i, nh - 1), 0)),
            pl.BlockSpec((IN_F, OUT_F), lambda i: (0, 0)),
            pl.BlockSpec((OUT_F, 1), lambda i: (0, 0)),
            pl.BlockSpec((OUT_F, 1), lambda i: (0, 0)),
        ],
        out_specs=[
            pl.BlockSpec((BF, OUT_F), lambda i: (i, 0)),
            pl.BlockSpec((BF, 1), lambda i: (jnp.minimum(i, nh - 1), 0)),
            pl.BlockSpec((BF, 1), lambda i: (jnp.minimum(i, nh - 1), 0)),
        ],
        out_shape=[
            jax.ShapeDtypeStruct((NPAD, OUT_F), jnp.float32),
            jax.ShapeDtypeStruct((N, 1), jnp.float32),
            jax.ShapeDtypeStruct((N, 1), jnp.float32),
        ],
    )(h, W, a1, a2)


def _kmeans_body(h_ref, dc_ref, w4_ref, utop_ref):
    TK = 1024

    def step(_, centers):
        c0 = centers[0:1, :]
        c1 = centers[1:2, :]
        c2 = centers[2:3, :]
        s0 = jnp.zeros((1, IN_F), jnp.float32)
        s1 = jnp.zeros((1, IN_F), jnp.float32)
        s2 = jnp.zeros((1, IN_F), jnp.float32)
        n0 = jnp.float32(0.0)
        n1 = jnp.float32(0.0)
        n2 = jnp.float32(0.0)
        for t in range(N // TK):
            ht = h_ref[pl.ds(t * TK, TK), :]

            def dist2(c_row):
                diff = ht - c_row
                return jnp.sum(diff * diff, axis=1, keepdims=True)

            d20, d21, d22 = dist2(c0), dist2(c1), dist2(c2)
            m01 = jnp.minimum(d20, d21)
            is2 = d22 < m01
            is1 = jnp.logical_and(jnp.logical_not(is2), d21 < d20)
            is0 = jnp.logical_and(jnp.logical_not(is2), jnp.logical_not(is1))
            s0 = s0 + jnp.sum(jnp.where(is0, ht, 0.0), axis=0, keepdims=True)
            s1 = s1 + jnp.sum(jnp.where(is1, ht, 0.0), axis=0, keepdims=True)
            s2 = s2 + jnp.sum(jnp.where(is2, ht, 0.0), axis=0, keepdims=True)
            n0 = n0 + jnp.sum(is0.astype(jnp.float32))
            n1 = n1 + jnp.sum(is1.astype(jnp.float32))
            n2 = n2 + jnp.sum(is2.astype(jnp.float32))
        return jnp.concatenate(
            [s0 / jnp.maximum(n0, 1.0),
             s1 / jnp.maximum(n1, 1.0),
             s2 / jnp.maximum(n2, 1.0),
             jnp.zeros((5, IN_F), jnp.float32)], axis=0)

    centers = jax.lax.fori_loop(0, 10, step, h_ref[0:8, :])
    h = h_ref[...]
    dc = dc_ref[0, 0]

    def dist1_full(c_row):
        diff = h - c_row
        return jnp.sqrt(jnp.sum(diff * diff, axis=1, keepdims=True))

    d1 = [dist1_full(centers[k:k + 1, :]) for k in range(KC)]
    row0 = jax.lax.broadcasted_iota(jnp.int32, (N, 1), 0) == 0
    near2 = jnp.where(row0, d1[1], d1[2])
    d1b = [jnp.where(dk != 0.0, dc * near2 / (dk * dk), dk) - 9e-15
           for dk in d1]
    ones = jnp.ones((N, 1), jnp.float32)
    zer = jnp.zeros((N, 1), jnp.float32)
    w4_ref[...] = jnp.concatenate(
        [ones, d1b[0], d1b[1], d1b[2], zer, zer, zer, zer], axis=1)
    utop_ref[...] = jnp.concatenate(
        [ones, zer, zer, zer, d1b[0], d1b[1], d1b[2], zer], axis=1)


def _kmeans_factors(h, dc2d):
    return pl.pallas_call(
        _kmeans_body,
        in_specs=[
            pl.BlockSpec((N, IN_F), lambda: (0, 0)),
            pl.BlockSpec((1, 1), lambda: (0, 0)),
        ],
        out_specs=[
            pl.BlockSpec((N, 8), lambda: (0, 0)),
            pl.BlockSpec((N, 8), lambda: (0, 0)),
        ],
        out_shape=[
            jax.ShapeDtypeStruct((N, 8), jnp.float32),
            jax.ShapeDtypeStruct((N, 8), jnp.float32),
        ],
    )(h, dc2d)


def _colsum_body(wh1_ref, wh2t_ref, w4_ref, t2p_ref, t1p_ref,
                 vmain_ref, acct_ref):
    BI = 512
    bn = wh2t_ref.shape[1]
    acc = jnp.zeros((8, bn), jnp.float32)
    wh2t = wh2t_ref[...]
    for i in range(N // BI):
        wh1t = wh1_ref[pl.ds(i * BI, BI), :]
        x = wh1t + wh2t
        e = jnp.where(x >= 0.0, x, ALPHA * x)
        w4t = w4_ref[pl.ds(i * BI, BI), :]
        acc = acc + jax.lax.dot_general(
            w4t, e, (((0,), (0,)), ((), ())),
            preferred_element_type=jnp.float32)
    t2p = t2p_ref[...]
    t2s = jnp.sum(t2p, axis=0, keepdims=True)
    r = jax.lax.broadcasted_iota(jnp.int32, (8, bn), 0)
    sel0 = r == 0
    sel13 = jnp.logical_and(r >= 1, r <= 3)
    vmain_ref[...] = (t2p
                      + jnp.where(sel0, jnp.float32(C_SMALL) * acc, 0.0)
                      + jnp.where(sel13, acc + jnp.float32(D_BIG) * t2s, 0.0))

    @pl.when(pl.program_id(0) == 0)
    def _():
        acct_ref[...] = jax.lax.dot_general(
            w4_ref[...], t1p_ref[...], (((0,), (0,)), ((), ())),
            preferred_element_type=jnp.float32)


def _colsums(wh1, wh2t, w4, t2pad, t1pad):
    BNC = 512
    return pl.pallas_call(
        _colsum_body,
        grid=(N // BNC,),
        in_specs=[
            pl.BlockSpec((N, 1), lambda j: (0, 0)),
            pl.BlockSpec((1, BNC), lambda j: (0, j)),
            pl.BlockSpec((N, 8), lambda j: (0, 0)),
            pl.BlockSpec((8, BNC), lambda j: (0, j)),
            pl.BlockSpec((N, 8), lambda j: (0, 0)),
        ],
        out_specs=[
            pl.BlockSpec((8, BNC), lambda j: (0, j)),
            pl.BlockSpec((8, 8), lambda j: (0, 0)),
        ],
        out_shape=[
            jax.ShapeDtypeStruct((8, N), jnp.float32),
            jax.ShapeDtypeStruct((8, 8), jnp.float32),
        ],
    )(wh1, wh2t, w4, t2pad, t1pad)


def _flash_body(u_ref, v_ref, whp_ref, out_ref, acc_ref, m_ref, l_ref):
    j = pl.program_id(1)
    nj = pl.num_programs(1)

    @pl.when(j == 0)
    def _():
        acc_ref[...] = jnp.zeros_like(acc_ref)
        m_ref[...] = jnp.full_like(m_ref, -jnp.inf)
        l_ref[...] = jnp.zeros_like(l_ref)

    v = v_ref[:, pl.ds(j * BF, BF)]
    whp = whp_ref[pl.ds(j * BF, BF), :]
    logits = jnp.dot(u_ref[...], v, preferred_element_type=jnp.float32)
    m_prev = m_ref[...]
    m_new = jnp.maximum(m_prev, jnp.max(logits, axis=1, keepdims=True))
    scale = jnp.exp(m_prev - m_new)
    p = jnp.exp(logits - m_new)
    l_ref[...] = l_ref[...] * scale + jnp.sum(p, axis=1, keepdims=True)
    acc_ref[...] = acc_ref[...] * scale + jnp.dot(
        p, whp, preferred_element_type=jnp.float32)
    m_ref[...] = m_new

    @pl.when(j == nj - 1)
    def _():
        hp = acc_ref[...] / l_ref[...]
        out_ref[...] = jnp.where(hp > 0.0, hp, jnp.exp(hp) - 1.0)


def _flash(u, v, whp):
    nb = NPAD // BF
    return pl.pallas_call(
        _flash_body,
        grid=(nb, nb),
        in_specs=[
            pl.BlockSpec((BF, 8), lambda i, j: (i, 0)),
            pl.BlockSpec((8, NPAD), lambda i, j: (0, 0)),
            pl.BlockSpec((NPAD, OUT_F), lambda i, j: (0, 0)),
        ],
        out_specs=pl.BlockSpec((BF, OUT_F), lambda i, j: (i, 0)),
        out_shape=jax.ShapeDtypeStruct((N + KC, OUT_F), jnp.float32),
        scratch_shapes=[
            pltpu.VMEM((BF, OUT_F), jnp.float32),
            pltpu.VMEM((BF, 1), jnp.float32),
            pltpu.VMEM((BF, 1), jnp.float32),
        ],
    )(u, v, whp)


def kernel(h, adj, W, a, tensor1, tensor2):
    del adj
    f32 = jnp.float32
    h = h.astype(f32)

    sq = jnp.sum(h * h, axis=1)
    m2 = _secondmin(h, sq.reshape(N, 1), sq.reshape(1, N))
    dc = jnp.mean(m2)

    w4, utop = _kmeans_factors(h, dc.reshape(1, 1))

    whp, wh1, wh2 = _proj(h, W, a[:OUT_F, :], a[OUT_F:, :])

    t2pad = jnp.concatenate(
        [jnp.zeros((4, N), f32), tensor2, jnp.zeros((1, N), f32)], axis=0)
    t1pad = jnp.concatenate(
        [tensor1, jnp.zeros((N, 8 - KC), f32)], axis=1)
    vmain, acct = _colsums(wh1, wh2.reshape(1, N), w4, t2pad, t1pad)

    npad_tail = NPAD - N - KC
    c = f32(C_SMALL)
    vtail_left = jnp.concatenate(
        [c * acct[0:1, :KC], acct[1:1 + KC, :KC],
         jnp.zeros((8 - 1 - KC, KC), f32)], axis=0)
    vtail_right = jnp.concatenate(
        [jnp.full((1, npad_tail), -1e30, f32),
         jnp.zeros((7, npad_tail), f32)], axis=0)
    v = jnp.concatenate([vmain, vtail_left, vtail_right], axis=1)

    ubot = jnp.zeros((NPAD - N, 8), f32)
    ubot = ubot.at[0:KC, 0].set(1.0)
    for k in range(KC):
        ubot = ubot.at[k, 1 + k].set(1.0)
    u = jnp.concatenate([utop, ubot], axis=0)

    return _flash(u, v, whp)


# speedup vs baseline: 37.9784x; 1.0632x over previous
import jax
import jax.numpy as jnp
from jax.experimental import pallas as pl
from jax.experimental.pallas import tpu as pltpu

N = 4096
IN_F = 512
OUT_F = 512
ALPHA = 0.2
KC = 3
C_SMALL = -9e-15
D_BIG = -9e15

NPAD = 4608
BM = 512
BF = 512
BFL = 768
BN1 = 1024


def _secondmin_body(hi_ref, hj_ref, sqi_ref, sqj_ref, out_ref, m1_ref, m2_ref):
    j = pl.program_id(1)
    nj = pl.num_programs(1)

    @pl.when(j == 0)
    def _():
        m1_ref[...] = jnp.full_like(m1_ref, jnp.inf)
        m2_ref[...] = jnp.full_like(m2_ref, jnp.inf)

    hj = hj_ref[pl.ds(j * BN1, BN1), :]
    sqj = sqj_ref[:, pl.ds(j * BN1, BN1)]
    dots = jax.lax.dot_general(
        hi_ref[...], hj, (((1,), (1,)), ((), ())),
        preferred_element_type=jnp.float32)
    d2 = sqi_ref[...] + sqj - 2.0 * dots
    d = jnp.sqrt(jnp.maximum(d2, 0.0))

    t1 = jnp.min(d, axis=1, keepdims=True)
    eq = d == t1
    cnt = jnp.sum(eq.astype(jnp.float32), axis=1, keepdims=True)
    t2_distinct = jnp.min(jnp.where(eq, jnp.inf, d), axis=1, keepdims=True)
    t2 = jnp.where(cnt > 1.0, t1, t2_distinct)

    r1 = m1_ref[...]
    r2 = m2_ref[...]
    m1_ref[...] = jnp.minimum(r1, t1)
    m2_ref[...] = jnp.minimum(jnp.maximum(r1, t1), jnp.minimum(r2, t2))

    @pl.when(j == nj - 1)
    def _():
        out_ref[...] = m2_ref[...]


def _secondmin(h, sq_col, sq_row):
    ni, nj = N // BM, N // BN1
    return pl.pallas_call(
        _secondmin_body,
        grid=(ni, nj),
        in_specs=[
            pl.BlockSpec((BM, IN_F), lambda i, j: (i, 0)),
            pl.BlockSpec((N, IN_F), lambda i, j: (0, 0)),
            pl.BlockSpec((BM, 1), lambda i, j: (i, 0)),
            pl.BlockSpec((1, N), lambda i, j: (0, 0)),
        ],
        out_specs=pl.BlockSpec((BM, 1), lambda i, j: (i, 0)),
        out_shape=jax.ShapeDtypeStruct((N, 1), jnp.float32),
        scratch_shapes=[
            pltpu.VMEM((BM, 1), jnp.float32),
            pltpu.VMEM((BM, 1), jnp.float32),
        ],
    )(h, h, sq_col, sq_row)


def _proj_body(h_ref, w_ref, a1_ref, a2_ref, wh_ref, wh1_ref, wh2_ref):
    i = pl.program_id(0)

    @pl.when(i < N // BF)
    def _():
        wh = jnp.dot(h_ref[...], w_ref[...], preferred_element_type=jnp.float32)
        wh_ref[...] = wh
        wh1_ref[...] = jnp.dot(wh, a1_ref[...],
                               preferred_element_type=jnp.float32)
        wh2_ref[...] = jnp.dot(wh, a2_ref[...],
                               preferred_element_type=jnp.float32)

    @pl.when(i >= N // BF)
    def _():
        wh_ref[...] = jnp.zeros_like(wh_ref)


def _proj(h, W, a1, a2):
    nh = N // BF
    return pl.pallas_call(
        _proj_body,
        grid=(NPAD // BF,),
        in_specs=[
            pl.BlockSpec((BF, IN_F), lambda i: (jnp.minimum(i, nh - 1), 0)),
            pl.BlockSpec((IN_F, OUT_F), lambda i: (0, 0)),
            pl.BlockSpec((OUT_F, 1), lambda i: (0, 0)),
            pl.BlockSpec((OUT_F, 1), lambda i: (0, 0)),
        ],
        out_specs=[
            pl.BlockSpec((BF, OUT_F), lambda i: (i, 0)),
            pl.BlockSpec((BF, 1), lambda i: (jnp.minimum(i, nh - 1), 0)),
            pl.BlockSpec((BF, 1), lambda i: (jnp.minimum(i, nh - 1), 0)),
        ],
        out_shape=[
            jax.ShapeDtypeStruct((NPAD, OUT_F), jnp.float32),
            jax.ShapeDtypeStruct((N, 1), jnp.float32),
            jax.ShapeDtypeStruct((N, 1), jnp.float32),
        ],
    )(h, W, a1, a2)


def _kmeans_body(h_ref, dc_ref, w4_ref, utop_ref):
    TK = 1024

    ones_f = jnp.ones((IN_F, 1), jnp.float32)
    ones_tk = jnp.ones((TK, 1), jnp.float32)

    def step(_, centers):
        c0 = centers[0:1, :]
        c1 = centers[1:2, :]
        c2 = centers[2:3, :]
        s_acc = jnp.zeros((8, IN_F), jnp.float32)
        n_acc = jnp.zeros((8, 1), jnp.float32)
        for t in range(N // TK):
            ht = h_ref[pl.ds(t * TK, TK), :]

            def dist2(c_row):
                diff = ht - c_row
                return jnp.dot(diff * diff, ones_f,
                               preferred_element_type=jnp.float32)

            d20, d21, d22 = dist2(c0), dist2(c1), dist2(c2)
            m01 = jnp.minimum(d20, d21)
            is2 = d22 < m01
            is1 = jnp.logical_and(jnp.logical_not(is2), d21 < d20)
            is0 = jnp.logical_and(jnp.logical_not(is2), jnp.logical_not(is1))
            zm = jnp.zeros((TK, 1), jnp.float32)
            masks8 = jnp.concatenate(
                [is0.astype(jnp.float32), is1.astype(jnp.float32),
                 is2.astype(jnp.float32), zm, zm, zm, zm, zm], axis=1)
            s_acc = s_acc + jax.lax.dot_general(
                masks8, ht, (((0,), (0,)), ((), ())),
                preferred_element_type=jnp.float32)
            n_acc = n_acc + jax.lax.dot_general(
                masks8, ones_tk, (((0,), (0,)), ((), ())),
                preferred_element_type=jnp.float32)
        n_clip = jnp.maximum(n_acc, 1.0)
        return jnp.concatenate(
            [s_acc[0:1, :] / n_clip[0:1, 0:1],
             s_acc[1:2, :] / n_clip[1:2, 0:1],
             s_acc[2:3, :] / n_clip[2:3, 0:1],
             jnp.zeros((5, IN_F), jnp.float32)], axis=0)

    centers = jax.lax.fori_loop(0, 10, step, h_ref[0:8, :])
    h = h_ref[...]
    dc = dc_ref[0, 0]

    ones_full = jnp.ones((IN_F, 1), jnp.float32)

    def dist1_full(c_row):
        diff = h - c_row
        return jnp.sqrt(jnp.dot(diff * diff, ones_full,
                                preferred_element_type=jnp.float32))

    d1 = [dist1_full(centers[k:k + 1, :]) for k in range(KC)]
    row0 = jax.lax.broadcasted_iota(jnp.int32, (N, 1), 0) == 0
    near2 = jnp.where(row0, d1[1], d1[2])
    d1b = [jnp.where(dk != 0.0, dc * near2 / (dk * dk), dk) - 9e-15
           for dk in d1]
    ones = jnp.ones((N, 1), jnp.float32)
    zer = jnp.zeros((N, 1), jnp.float32)
    w4_ref[...] = jnp.concatenate(
        [ones, d1b[0], d1b[1], d1b[2], zer, zer, zer, zer], axis=1)
    utop_ref[...] = jnp.concatenate(
        [ones, zer, zer, zer, d1b[0], d1b[1], d1b[2], zer], axis=1)


def _kmeans_factors(h, dc2d):
    return pl.pallas_call(
        _kmeans_body,
        in_specs=[
            pl.BlockSpec((N, IN_F), lambda: (0, 0)),
            pl.BlockSpec((1, 1), lambda: (0, 0)),
        ],
        out_specs=[
            pl.BlockSpec((N, 8), lambda: (0, 0)),
            pl.BlockSpec((N, 8), lambda: (0, 0)),
        ],
        out_shape=[
            jax.ShapeDtypeStruct((N, 8), jnp.float32),
            jax.ShapeDtypeStruct((N, 8), jnp.float32),
        ],
    )(h, dc2d)


def _colsum_body(wh1_ref, wh2t_ref, w4_ref, t2p_ref, t1p_ref,
                 vmain_ref, acct_ref):
    BI = 512
    bn = wh2t_ref.shape[1]
    acc = jnp.zeros((8, bn), jnp.float32)
    wh2t = wh2t_ref[...]
    for i in range(N // BI):
        wh1t = wh1_ref[pl.ds(i * BI, BI), :]
        x = wh1t + wh2t
        e = jnp.where(x >= 0.0, x, ALPHA * x)
        w4t = w4_ref[pl.ds(i * BI, BI), :]
        acc = acc + jax.lax.dot_general(
            w4t, e, (((0,), (0,)), ((), ())),
            preferred_element_type=jnp.float32)
    t2p = t2p_ref[...]
    t2s = jnp.sum(t2p, axis=0, keepdims=True)
    r = jax.lax.broadcasted_iota(jnp.int32, (8, bn), 0)
    sel0 = r == 0
    sel13 = jnp.logical_and(r >= 1, r <= 3)
    vmain_ref[...] = (t2p
                      + jnp.where(sel0, jnp.float32(C_SMALL) * acc, 0.0)
                      + jnp.where(sel13, acc + jnp.float32(D_BIG) * t2s, 0.0))

    @pl.when(pl.program_id(0) == 0)
    def _():
        acct_ref[...] = jax.lax.dot_general(
            w4_ref[...], t1p_ref[...], (((0,), (0,)), ((), ())),
            preferred_element_type=jnp.float32)


def _colsums(wh1, wh2t, w4, t2pad, t1pad):
    BNC = 512
    return pl.pallas_call(
        _colsum_body,
        grid=(N // BNC,),
        in_specs=[
            pl.BlockSpec((N, 1), lambda j: (0, 0)),
            pl.BlockSpec((1, BNC), lambda j: (0, j)),
            pl.BlockSpec((N, 8), lambda j: (0, 0)),
            pl.BlockSpec((8, BNC), lambda j: (0, j)),
            pl.BlockSpec((N, 8), lambda j: (0, 0)),
        ],
        out_specs=[
            pl.BlockSpec((8, BNC), lambda j: (0, j)),
            pl.BlockSpec((8, 8), lambda j: (0, 0)),
        ],
        out_shape=[
            jax.ShapeDtypeStruct((8, N), jnp.float32),
            jax.ShapeDtypeStruct((8, 8), jnp.float32),
        ],
    )(wh1, wh2t, w4, t2pad, t1pad)


def _flash_body(u_ref, v_ref, whp_ref, out_ref, acc_ref, m_ref, l_ref):
    j = pl.program_id(1)
    nj = pl.num_programs(1)

    @pl.when(j == 0)
    def _():
        acc_ref[...] = jnp.zeros_like(acc_ref)
        m_ref[...] = jnp.full_like(m_ref, -jnp.inf)
        l_ref[...] = jnp.zeros_like(l_ref)

    v = v_ref[:, pl.ds(j * BFL, BFL)]
    whp = whp_ref[pl.ds(j * BFL, BFL), :]
    logits = jnp.dot(u_ref[...], v, preferred_element_type=jnp.float32)
    m_prev = m_ref[...]
    m_new = jnp.maximum(m_prev, jnp.max(logits, axis=1, keepdims=True))
    scale = jnp.exp(m_prev - m_new)
    p = jnp.exp(logits - m_new)
    l_ref[...] = l_ref[...] * scale + jnp.sum(p, axis=1, keepdims=True)
    acc_ref[...] = acc_ref[...] * scale + jnp.dot(
        p, whp, preferred_element_type=jnp.float32)
    m_ref[...] = m_new

    @pl.when(j == nj - 1)
    def _():
        hp = acc_ref[...] / l_ref[...]
        out_ref[...] = jnp.where(hp > 0.0, hp, jnp.exp(hp) - 1.0)


def _flash(u, v, whp):
    nb = NPAD // BFL
    return pl.pallas_call(
        _flash_body,
        grid=(nb, nb),
        in_specs=[
            pl.BlockSpec((BFL, 8), lambda i, j: (i, 0)),
            pl.BlockSpec((8, NPAD), lambda i, j: (0, 0)),
            pl.BlockSpec((NPAD, OUT_F), lambda i, j: (0, 0)),
        ],
        out_specs=pl.BlockSpec((BFL, OUT_F), lambda i, j: (i, 0)),
        out_shape=jax.ShapeDtypeStruct((N + KC, OUT_F), jnp.float32),
        scratch_shapes=[
            pltpu.VMEM((BFL, OUT_F), jnp.float32),
            pltpu.VMEM((BFL, 1), jnp.float32),
            pltpu.VMEM((BFL, 1), jnp.float32),
        ],
    )(u, v, whp)


def kernel(h, adj, W, a, tensor1, tensor2):
    del adj
    f32 = jnp.float32
    h = h.astype(f32)

    sq = jnp.sum(h * h, axis=1)
    m2 = _secondmin(h, sq.reshape(N, 1), sq.reshape(1, N))
    dc = jnp.mean(m2)

    w4, utop = _kmeans_factors(h, dc.reshape(1, 1))

    whp, wh1, wh2 = _proj(h, W, a[:OUT_F, :], a[OUT_F:, :])

    t2pad = jnp.concatenate(
        [jnp.zeros((4, N), f32), tensor2, jnp.zeros((1, N), f32)], axis=0)
    t1pad = jnp.concatenate(
        [tensor1, jnp.zeros((N, 8 - KC), f32)], axis=1)
    vmain, acct = _colsums(wh1, wh2.reshape(1, N), w4, t2pad, t1pad)

    npad_tail = NPAD - N - KC
    c = f32(C_SMALL)
    vtail_left = jnp.concatenate(
        [c * acct[0:1, :KC], acct[1:1 + KC, :KC],
         jnp.zeros((8 - 1 - KC, KC), f32)], axis=0)
    vtail_right = jnp.concatenate(
        [jnp.full((1, npad_tail), -1e30, f32),
         jnp.zeros((7, npad_tail), f32)], axis=0)
    v = jnp.concatenate([vmain, vtail_left, vtail_right], axis=1)

    ubot = jnp.zeros((NPAD - N, 8), f32)
    ubot = ubot.at[0:KC, 0].set(1.0)
    for k in range(KC):
        ubot = ubot.at[k, 1 + k].set(1.0)
    u = jnp.concatenate([utop, ubot], axis=0)

    return _flash(u, v, whp)


# speedup vs baseline: 42.3867x; 1.1161x over previous
import jax
import jax.numpy as jnp
from jax.experimental import pallas as pl
from jax.experimental.pallas import tpu as pltpu

N = 4096
IN_F = 512
OUT_F = 512
ALPHA = 0.2
KC = 3
C_SMALL = -9e-15
D_BIG = -9e15

NPAD = 4608
BM = 512
BF = 512
BFL = 768
BN1 = 1024


def _secondmin_body(hi_ref, hj_ref, sqi_ref, sqj_ref, out_ref, m1_ref, m2_ref):
    j = pl.program_id(1)
    nj = pl.num_programs(1)

    @pl.when(j == 0)
    def _():
        m1_ref[...] = jnp.full_like(m1_ref, jnp.inf)
        m2_ref[...] = jnp.full_like(m2_ref, jnp.inf)

    hj = hj_ref[pl.ds(j * BN1, BN1), :]
    sqj = sqj_ref[:, pl.ds(j * BN1, BN1)]
    dots = jax.lax.dot_general(
        hi_ref[...], hj, (((1,), (1,)), ((), ())),
        preferred_element_type=jnp.float32)
    d = jnp.maximum(sqi_ref[...] + sqj - 2.0 * dots, 0.0)

    t1 = jnp.min(d, axis=1, keepdims=True)
    eq = d == t1
    cnt = jnp.sum(eq.astype(jnp.float32), axis=1, keepdims=True)
    t2_distinct = jnp.min(jnp.where(eq, jnp.inf, d), axis=1, keepdims=True)
    t2 = jnp.where(cnt > 1.0, t1, t2_distinct)

    r1 = m1_ref[...]
    r2 = m2_ref[...]
    m1_ref[...] = jnp.minimum(r1, t1)
    m2_ref[...] = jnp.minimum(jnp.maximum(r1, t1), jnp.minimum(r2, t2))

    @pl.when(j == nj - 1)
    def _():
        out_ref[...] = jnp.sqrt(m2_ref[...])


def _secondmin(h, sq_col, sq_row):
    ni, nj = N // BM, N // BN1
    return pl.pallas_call(
        _secondmin_body,
        grid=(ni, nj),
        in_specs=[
            pl.BlockSpec((BM, IN_F), lambda i, j: (i, 0)),
            pl.BlockSpec((N, IN_F), lambda i, j: (0, 0)),
            pl.BlockSpec((BM, 1), lambda i, j: (i, 0)),
            pl.BlockSpec((1, N), lambda i, j: (0, 0)),
        ],
        out_specs=pl.BlockSpec((BM, 1), lambda i, j: (i, 0)),
        out_shape=jax.ShapeDtypeStruct((N, 1), jnp.float32),
        scratch_shapes=[
            pltpu.VMEM((BM, 1), jnp.float32),
            pltpu.VMEM((BM, 1), jnp.float32),
        ],
    )(h, h, sq_col, sq_row)


def _proj_body(h_ref, w_ref, a1_ref, a2_ref, wh_ref, wh1_ref, wh2_ref):
    i = pl.program_id(0)

    @pl.when(i < N // BF)
    def _():
        wh = jnp.dot(h_ref[...], w_ref[...], preferred_element_type=jnp.float32)
        wh_ref[...] = wh
        wh1_ref[...] = jnp.dot(wh, a1_ref[...],
                               preferred_element_type=jnp.float32)
        wh2_ref[...] = jnp.dot(wh, a2_ref[...],
                               preferred_element_type=jnp.float32)

    @pl.when(i >= N // BF)
    def _():
        wh_ref[...] = jnp.zeros_like(wh_ref)


def _proj(h, W, a1, a2):
    nh = N // BF
    return pl.pallas_call(
        _proj_body,
        grid=(NPAD // BF,),
        in_specs=[
            pl.BlockSpec((BF, IN_F), lambda i: (jnp.minimum(i, nh - 1), 0)),
            pl.BlockSpec((IN_F, OUT_F), lambda i: (0, 0)),
            pl.BlockSpec((OUT_F, 1), lambda i: (0, 0)),
            pl.BlockSpec((OUT_F, 1), lambda i: (0, 0)),
        ],
        out_specs=[
            pl.BlockSpec((BF, OUT_F), lambda i: (i, 0)),
            pl.BlockSpec((BF, 1), lambda i: (jnp.minimum(i, nh - 1), 0)),
            pl.BlockSpec((BF, 1), lambda i: (jnp.minimum(i, nh - 1), 0)),
        ],
        out_shape=[
            jax.ShapeDtypeStruct((NPAD, OUT_F), jnp.float32),
            jax.ShapeDtypeStruct((N, 1), jnp.float32),
            jax.ShapeDtypeStruct((N, 1), jnp.float32),
        ],
    )(h, W, a1, a2)


def _kmeans_body(h_ref, dc_ref, w4_ref, utop_ref):
    TK = 4096

    def step(_, centers):
        c0 = centers[0:1, :]
        c1 = centers[1:2, :]
        c2 = centers[2:3, :]
        s0 = jnp.zeros((1, IN_F), jnp.float32)
        s1 = jnp.zeros((1, IN_F), jnp.float32)
        s2 = jnp.zeros((1, IN_F), jnp.float32)
        n0 = jnp.float32(0.0)
        n1 = jnp.float32(0.0)
        n2 = jnp.float32(0.0)
        for t in range(N // TK):
            ht = h_ref[pl.ds(t * TK, TK), :]

            def dist2(c_row):
                diff = ht - c_row
                return jnp.sum(diff * diff, axis=1, keepdims=True)

            d20, d21, d22 = dist2(c0), dist2(c1), dist2(c2)
            m01 = jnp.minimum(d20, d21)
            is2 = d22 < m01
            is1 = jnp.logical_and(jnp.logical_not(is2), d21 < d20)
            is0 = jnp.logical_and(jnp.logical_not(is2), jnp.logical_not(is1))
            s0 = s0 + jnp.sum(jnp.where(is0, ht, 0.0), axis=0, keepdims=True)
            s1 = s1 + jnp.sum(jnp.where(is1, ht, 0.0), axis=0, keepdims=True)
            s2 = s2 + jnp.sum(jnp.where(is2, ht, 0.0), axis=0, keepdims=True)
            n0 = n0 + jnp.sum(is0.astype(jnp.float32))
            n1 = n1 + jnp.sum(is1.astype(jnp.float32))
            n2 = n2 + jnp.sum(is2.astype(jnp.float32))
        return jnp.concatenate(
            [s0 / jnp.maximum(n0, 1.0),
             s1 / jnp.maximum(n1, 1.0),
             s2 / jnp.maximum(n2, 1.0),
             jnp.zeros((5, IN_F), jnp.float32)], axis=0)

    centers = jax.lax.fori_loop(0, 10, step, h_ref[0:8, :])
    h = h_ref[...]
    dc = dc_ref[0, 0]

    def dist1_full(c_row):
        diff = h - c_row
        return jnp.sqrt(jnp.sum(diff * diff, axis=1, keepdims=True))

    d1 = [dist1_full(centers[k:k + 1, :]) for k in range(KC)]
    row0 = jax.lax.broadcasted_iota(jnp.int32, (N, 1), 0) == 0
    near2 = jnp.where(row0, d1[1], d1[2])
    d1b = [jnp.where(dk != 0.0, dc * near2 / (dk * dk), dk) - 9e-15
           for dk in d1]
    ones = jnp.ones((N, 1), jnp.float32)
    zer = jnp.zeros((N, 1), jnp.float32)
    w4_ref[...] = jnp.concatenate(
        [ones, d1b[0], d1b[1], d1b[2], zer, zer, zer, zer], axis=1)
    utop_ref[...] = jnp.concatenate(
        [ones, zer, zer, zer, d1b[0], d1b[1], d1b[2], zer], axis=1)


def _kmeans_factors(h, dc2d):
    return pl.pallas_call(
        _kmeans_body,
        in_specs=[
            pl.BlockSpec((N, IN_F), lambda: (0, 0)),
            pl.BlockSpec((1, 1), lambda: (0, 0)),
        ],
        out_specs=[
            pl.BlockSpec((N, 8), lambda: (0, 0)),
            pl.BlockSpec((N, 8), lambda: (0, 0)),
        ],
        out_shape=[
            jax.ShapeDtypeStruct((N, 8), jnp.float32),
            jax.ShapeDtypeStruct((N, 8), jnp.float32),
        ],
    )(h, dc2d)


def _colsum_body(wh1_ref, wh2t_ref, w4_ref, t2p_ref, t1p_ref,
                 vmain_ref, acct_ref):
    BI = 512
    bn = wh2t_ref.shape[1]
    acc = jnp.zeros((8, bn), jnp.float32)
    wh2t = wh2t_ref[...]
    for i in range(N // BI):
        wh1t = wh1_ref[pl.ds(i * BI, BI), :]
        x = wh1t + wh2t
        e = jnp.where(x >= 0.0, x, ALPHA * x)
        w4t = w4_ref[pl.ds(i * BI, BI), :]
        acc = acc + jax.lax.dot_general(
            w4t, e, (((0,), (0,)), ((), ())),
            preferred_element_type=jnp.float32)
    t2p = t2p_ref[...]
    t2s = jnp.sum(t2p, axis=0, keepdims=True)
    r = jax.lax.broadcasted_iota(jnp.int32, (8, bn), 0)
    sel0 = r == 0
    sel13 = jnp.logical_and(r >= 1, r <= 3)
    vmain_ref[...] = (t2p
                      + jnp.where(sel0, jnp.float32(C_SMALL) * acc, 0.0)
                      + jnp.where(sel13, acc + jnp.float32(D_BIG) * t2s, 0.0))

    @pl.when(pl.program_id(0) == 0)
    def _():
        acct_ref[...] = jax.lax.dot_general(
            w4_ref[...], t1p_ref[...], (((0,), (0,)), ((), ())),
            preferred_element_type=jnp.float32)


def _colsums(wh1, wh2t, w4, t2pad, t1pad):
    BNC = 512
    return pl.pallas_call(
        _colsum_body,
        grid=(N // BNC,),
        in_specs=[
            pl.BlockSpec((N, 1), lambda j: (0, 0)),
            pl.BlockSpec((1, BNC), lambda j: (0, j)),
            pl.BlockSpec((N, 8), lambda j: (0, 0)),
            pl.BlockSpec((8, BNC), lambda j: (0, j)),
            pl.BlockSpec((N, 8), lambda j: (0, 0)),
        ],
        out_specs=[
            pl.BlockSpec((8, BNC), lambda j: (0, j)),
            pl.BlockSpec((8, 8), lambda j: (0, 0)),
        ],
        out_shape=[
            jax.ShapeDtypeStruct((8, N), jnp.float32),
            jax.ShapeDtypeStruct((8, 8), jnp.float32),
        ],
    )(wh1, wh2t, w4, t2pad, t1pad)


def _flash_body(u_ref, v_ref, whp_ref, out_ref, acc_ref, m_ref, l_ref):
    j = pl.program_id(1)
    nj = pl.num_programs(1)

    @pl.when(j == 0)
    def _():
        acc_ref[...] = jnp.zeros_like(acc_ref)
        m_ref[...] = jnp.full_like(m_ref, -jnp.inf)
        l_ref[...] = jnp.zeros_like(l_ref)

    v = v_ref[:, pl.ds(j * BFL, BFL)]
    whp = whp_ref[pl.ds(j * BFL, BFL), :]
    logits = jnp.dot(u_ref[...], v, preferred_element_type=jnp.float32)
    m_prev = m_ref[...]
    m_new = jnp.maximum(m_prev, jnp.max(logits, axis=1, keepdims=True))
    scale = jnp.exp(m_prev - m_new)
    p = jnp.exp(logits - m_new)
    l_ref[...] = l_ref[...] * scale + jnp.sum(p, axis=1, keepdims=True)
    acc_ref[...] = acc_ref[...] * scale + jnp.dot(
        p, whp, preferred_element_type=jnp.float32)
    m_ref[...] = m_new

    @pl.when(j == nj - 1)
    def _():
        hp = acc_ref[...] / l_ref[...]
        out_ref[...] = jnp.where(hp > 0.0, hp, jnp.exp(hp) - 1.0)


def _flash(u, v, whp):
    nb = NPAD // BFL
    return pl.pallas_call(
        _flash_body,
        grid=(nb, nb),
        in_specs=[
            pl.BlockSpec((BFL, 8), lambda i, j: (i, 0)),
            pl.BlockSpec((8, NPAD), lambda i, j: (0, 0)),
            pl.BlockSpec((NPAD, OUT_F), lambda i, j: (0, 0)),
        ],
        out_specs=pl.BlockSpec((BFL, OUT_F), lambda i, j: (i, 0)),
        out_shape=jax.ShapeDtypeStruct((N + KC, OUT_F), jnp.float32),
        scratch_shapes=[
            pltpu.VMEM((BFL, OUT_F), jnp.float32),
            pltpu.VMEM((BFL, 1), jnp.float32),
            pltpu.VMEM((BFL, 1), jnp.float32),
        ],
    )(u, v, whp)


def kernel(h, adj, W, a, tensor1, tensor2):
    del adj
    f32 = jnp.float32
    h = h.astype(f32)

    sq = jnp.sum(h * h, axis=1)
    m2 = _secondmin(h, sq.reshape(N, 1), sq.reshape(1, N))
    dc = jnp.mean(m2)

    w4, utop = _kmeans_factors(h, dc.reshape(1, 1))

    whp, wh1, wh2 = _proj(h, W, a[:OUT_F, :], a[OUT_F:, :])

    t2pad = jnp.concatenate(
        [jnp.zeros((4, N), f32), tensor2, jnp.zeros((1, N), f32)], axis=0)
    t1pad = jnp.concatenate(
        [tensor1, jnp.zeros((N, 8 - KC), f32)], axis=1)
    vmain, acct = _colsums(wh1, wh2.reshape(1, N), w4, t2pad, t1pad)

    npad_tail = NPAD - N - KC
    c = f32(C_SMALL)
    vtail_left = jnp.concatenate(
        [c * acct[0:1, :KC], acct[1:1 + KC, :KC],
         jnp.zeros((8 - 1 - KC, KC), f32)], axis=0)
    vtail_right = jnp.concatenate(
        [jnp.full((1, npad_tail), -1e30, f32),
         jnp.zeros((7, npad_tail), f32)], axis=0)
    v = jnp.concatenate([vmain, vtail_left, vtail_right], axis=1)

    ubot = jnp.zeros((NPAD - N, 8), f32)
    ubot = ubot.at[0:KC, 0].set(1.0)
    for k in range(KC):
        ubot = ubot.at[k, 1 + k].set(1.0)
    u = jnp.concatenate([utop, ubot], axis=0)

    return _flash(u, v, whp)
